# baseline (speedup 1.0000x reference)
import functools

import jax
import jax.numpy as jnp
from jax import lax
from jax.experimental import pallas as pl
from jax.experimental.pallas import tpu as pltpu
from jax.experimental.pallas import tpu_sc as plsc

N = 10000
E = 320000
D = 128

NC = 2
NS = 16
CHUNK = 40
EPC = E // NC
EPS = EPC // NS
NCHUNK = EPS // CHUNK
NPS = 624
NBUF = 4
NTAIL = N - NS * NPS
SROWS = 48

_MESH = plsc.VectorSubcoreMesh(core_axis_name="c", subcore_axis_name="s",
                               num_cores=NC, num_subcores=NS)


def _zero_fill(buf, rows):
    def zf(i, carry):
        for g in range(D // 16):
            buf[i, pl.ds(g * 16, 16)] = jnp.zeros((16,), jnp.float32)
        return carry
    lax.fori_loop(0, rows, zf, 0)


def _zero_shared(stage_v, sh, s):
    nbase = s * NPS
    for k in range(NPS // SROWS):
        pltpu.sync_copy(stage_v, sh.at[pl.ds(nbase + k * SROWS, SROWS)])

    @pl.when(s == NS - 1)
    def _():
        pltpu.sync_copy(stage_v.at[pl.ds(0, NTAIL)],
                        sh.at[pl.ds(NS * NPS, NTAIL)])


def _write_back(stage_v, sh, out, c, s):
    nbase = s * NPS
    obase = c * N + nbase
    for k in range(NPS // SROWS):
        pltpu.sync_copy(sh.at[pl.ds(nbase + k * SROWS, SROWS)], stage_v)
        pltpu.sync_copy(stage_v, out.at[pl.ds(obase + k * SROWS, SROWS)])

    @pl.when(s == NS - 1)
    def _():
        pltpu.sync_copy(sh.at[pl.ds(NS * NPS, NTAIL)],
                        stage_v.at[pl.ds(0, NTAIL)])
        pltpu.sync_copy(stage_v.at[pl.ds(0, NTAIL)],
                        out.at[pl.ds(c * N + NS * NPS, NTAIL)])


@functools.partial(
    pl.kernel,
    out_type=jax.ShapeDtypeStruct((NC * N, D), jnp.float32),
    mesh=_MESH,
    scratch_types=(
        tuple(pltpu.VMEM((CHUNK,), jnp.int32) for _ in range(NBUF)),
        tuple(pltpu.VMEM((CHUNK,), jnp.int32) for _ in range(NBUF)),
        tuple(pltpu.VMEM((CHUNK, D), jnp.float32) for _ in range(NBUF)),
        pltpu.VMEM((SROWS, D), jnp.float32),
        pltpu.VMEM_SHARED((N, D), jnp.float32),
        tuple(pltpu.SemaphoreType.DMA for _ in range(NBUF)),
        tuple(pltpu.SemaphoreType.DMA for _ in range(NBUF)),
    ))
def _sc_agg(src_hbm, dst_hbm, m_hbm, agg_out,
            srcb, dstb, rowb, stage_v, agg_sh, gsem, ssem):
    c = lax.axis_index("c")
    s = lax.axis_index("s")
    ebase = c * EPC + s * EPS
    NB = NBUF

    _zero_fill(stage_v, SROWS)
    _zero_shared(stage_v, agg_sh, s)
    plsc.subcore_barrier()

    def _load_fire(j, b):
        base = pl.multiple_of(ebase + j * CHUNK, 8)
        pltpu.sync_copy(src_hbm.at[pl.ds(base, CHUNK)], srcb[b])
        pltpu.sync_copy(dst_hbm.at[pl.ds(base, CHUNK)], dstb[b])
        pltpu.async_copy(m_hbm.at[srcb[b]], rowb[b], gsem[b])

    for b in range(NB):
        _load_fire(b, b)

    def step(i, carry):
        for b in range(NB):
            pltpu.make_async_copy(m_hbm.at[srcb[b]], rowb[b], gsem[b]).wait()
            pltpu.async_copy(rowb[b], agg_sh.at[dstb[b]], ssem[b], add=True)
        for b in range(NB):
            pltpu.make_async_copy(rowb[b], agg_sh.at[dstb[b]], ssem[b]).wait()
            nj = i * NB + b + NB

            @pl.when(nj < NCHUNK)
            def _():
                _load_fire(nj, b)
        return carry

    lax.fori_loop(0, NCHUNK // NB, step, 0)
    for t in range(NCHUNK % NB):
        pltpu.make_async_copy(m_hbm.at[srcb[t]], rowb[t], gsem[t]).wait()
        pltpu.sync_copy(rowb[t], agg_sh.at[dstb[t]], add=True)
    plsc.subcore_barrier()
    _write_back(stage_v, agg_sh, agg_out, c, s)


@functools.partial(
    pl.kernel,
    out_type=jax.ShapeDtypeStruct((NC * N, D), jnp.float32),
    mesh=_MESH,
    scratch_types=(
        tuple(pltpu.VMEM((CHUNK,), jnp.int32) for _ in range(NBUF)),
        pltpu.VMEM((CHUNK, D), jnp.float32),
        pltpu.VMEM((SROWS, D), jnp.float32),
        pltpu.VMEM_SHARED((N, D), jnp.float32),
        tuple(pltpu.SemaphoreType.DMA for _ in range(NBUF)),
    ))
def _sc_deg(dst_hbm, deg_out, dstb, ones_v, stage_v, deg_sh, ssem):
    c = lax.axis_index("c")
    s = lax.axis_index("s")
    ebase = c * EPC + s * EPS
    NB = NBUF

    def of(i, carry):
        for g in range(D // 16):
            ones_v[i, pl.ds(g * 16, 16)] = jnp.ones((16,), jnp.float32)
        return carry
    lax.fori_loop(0, CHUNK, of, 0)

    _zero_fill(stage_v, SROWS)
    _zero_shared(stage_v, deg_sh, s)
    plsc.subcore_barrier()

    def _load_fire(j, b):
        base = pl.multiple_of(ebase + j * CHUNK, 8)
        pltpu.sync_copy(dst_hbm.at[pl.ds(base, CHUNK)], dstb[b])
        pltpu.async_copy(ones_v, deg_sh.at[dstb[b]], ssem[b], add=True)

    for b in range(NB):
        _load_fire(b, b)

    def step(i, carry):
        for b in range(NB):
            pltpu.make_async_copy(ones_v, deg_sh.at[dstb[b]], ssem[b]).wait()
            nj = i * NB + b + NB

            @pl.when(nj < NCHUNK)
            def _():
                _load_fire(nj, b)
        return carry

    lax.fori_loop(0, NCHUNK // NB, step, 0)
    for t in range(NCHUNK % NB):
        pltpu.make_async_copy(ones_v, deg_sh.at[dstb[t]], ssem[t]).wait()
    plsc.subcore_barrier()
    _write_back(stage_v, deg_sh, deg_out, c, s)


BR = 2000
GB = N // BR


def _mm_body(x_ref, w_ref, b_ref, m_ref):
    m_ref[...] = (jnp.dot(x_ref[...], w_ref[...],
                          preferred_element_type=jnp.float32) + b_ref[...])


def _tc_mm(x, W, b):
    return pl.pallas_call(
        _mm_body,
        grid=(GB,),
        in_specs=[pl.BlockSpec((BR, D), lambda i: (i, 0)),
                  pl.BlockSpec((D, D), lambda i: (0, 0)),
                  pl.BlockSpec((1, D), lambda i: (0, 0))],
        out_specs=pl.BlockSpec((BR, D), lambda i: (i, 0)),
        out_shape=jax.ShapeDtypeStruct((N, D), jnp.float32),
    )(x, W, b.reshape(1, D))


def _combine(h_ref, a_ref, g_ref):
    agg = a_ref[0] + a_ref[1]
    deg = jnp.maximum(g_ref[0, :, 0:1] + g_ref[1, :, 0:1], 1.0)
    return h_ref[...] + jnp.maximum(agg / deg, 0.0)


def _combine_mm_body(h_ref, a_ref, g_ref, w_ref, b_ref, hn_ref, m_ref):
    hn = _combine(h_ref, a_ref, g_ref)
    hn_ref[...] = hn
    m_ref[...] = (jnp.dot(hn, w_ref[...],
                          preferred_element_type=jnp.float32) + b_ref[...])


def _tc_combine_mm(h, agg2, deg2, W, b):
    return pl.pallas_call(
        _combine_mm_body,
        grid=(GB,),
        in_specs=[pl.BlockSpec((BR, D), lambda i: (i, 0)),
                  pl.BlockSpec((2, BR, D), lambda i: (0, i, 0)),
                  pl.BlockSpec((2, BR, D), lambda i: (0, i, 0)),
                  pl.BlockSpec((D, D), lambda i: (0, 0)),
                  pl.BlockSpec((1, D), lambda i: (0, 0))],
        out_specs=[pl.BlockSpec((BR, D), lambda i: (i, 0)),
                   pl.BlockSpec((BR, D), lambda i: (i, 0))],
        out_shape=[jax.ShapeDtypeStruct((N, D), jnp.float32),
                   jax.ShapeDtypeStruct((N, D), jnp.float32)],
    )(h, agg2, deg2, W, b.reshape(1, D))


def _combine_norm_body(h_ref, a_ref, g_ref, o_ref):
    hn = _combine(h_ref, a_ref, g_ref)
    nrm = jnp.sqrt(jnp.sum(hn * hn, axis=-1, keepdims=True))
    o_ref[...] = hn / jnp.maximum(nrm, 1e-12)


def _tc_combine_norm(h, agg2, deg2):
    return pl.pallas_call(
        _combine_norm_body,
        grid=(GB,),
        in_specs=[pl.BlockSpec((BR, D), lambda i: (i, 0)),
                  pl.BlockSpec((2, BR, D), lambda i: (0, i, 0)),
                  pl.BlockSpec((2, BR, D), lambda i: (0, i, 0))],
        out_specs=pl.BlockSpec((BR, D), lambda i: (i, 0)),
        out_shape=jax.ShapeDtypeStruct((N, D), jnp.float32),
    )(h, agg2, deg2)


def kernel(x, edge_index, W0, b0, W1, b1, W2, b2):
    src = edge_index[0]
    dst = edge_index[1]

    deg2 = _sc_deg(dst).reshape(2, N, D)
    m0 = _tc_mm(x, W0, b0)
    agg0 = _sc_agg(src, dst, m0).reshape(2, N, D)

    h1, m1 = _tc_combine_mm(x, agg0, deg2, W1, b1)
    agg1 = _sc_agg(src, dst, m1).reshape(2, N, D)

    h2, m2 = _tc_combine_mm(h1, agg1, deg2, W2, b2)
    agg2 = _sc_agg(src, dst, m2).reshape(2, N, D)

    return _tc_combine_norm(h2, agg2, deg2)


# speedup vs baseline: 1.5669x; 1.5669x over previous
import functools

import jax
import jax.numpy as jnp
from jax import lax
from jax.experimental import pallas as pl
from jax.experimental.pallas import tpu as pltpu
from jax.experimental.pallas import tpu_sc as plsc

N = 10000
E = 320000
D = 128

NC = 2
NS = 16
CHUNK = 80
EPC = E // NC
EPS = EPC // NS
NCHUNK = EPS // CHUNK
NPS = 624
NBUF = 4
NTAIL = N - NS * NPS
SROWS = 16

_MESH = plsc.VectorSubcoreMesh(core_axis_name="c", subcore_axis_name="s",
                               num_cores=NC, num_subcores=NS)


def _zero_fill(buf, rows):
    def zf(i, carry):
        for g in range(D // 16):
            buf[i, pl.ds(g * 16, 16)] = jnp.zeros((16,), jnp.float32)
        return carry
    lax.fori_loop(0, rows, zf, 0)


def _zero_shared(stage_v, sh, s):
    nbase = s * NPS
    for k in range(NPS // SROWS):
        pltpu.sync_copy(stage_v, sh.at[pl.ds(nbase + k * SROWS, SROWS)])

    @pl.when(s == NS - 1)
    def _():
        pltpu.sync_copy(stage_v.at[pl.ds(0, NTAIL)],
                        sh.at[pl.ds(NS * NPS, NTAIL)])


def _write_back(stage_v, sh, out, c, s):
    nbase = s * NPS
    obase = c * N + nbase
    for k in range(NPS // SROWS):
        pltpu.sync_copy(sh.at[pl.ds(nbase + k * SROWS, SROWS)], stage_v)
        pltpu.sync_copy(stage_v, out.at[pl.ds(obase + k * SROWS, SROWS)])

    @pl.when(s == NS - 1)
    def _():
        pltpu.sync_copy(sh.at[pl.ds(NS * NPS, NTAIL)],
                        stage_v.at[pl.ds(0, NTAIL)])
        pltpu.sync_copy(stage_v.at[pl.ds(0, NTAIL)],
                        out.at[pl.ds(c * N + NS * NPS, NTAIL)])


@functools.partial(
    pl.kernel,
    out_type=jax.ShapeDtypeStruct((NC * N, D), jnp.float32),
    mesh=_MESH,
    scratch_types=(
        tuple(pltpu.VMEM((CHUNK,), jnp.int32) for _ in range(NBUF)),
        tuple(pltpu.VMEM((CHUNK,), jnp.int32) for _ in range(NBUF)),
        tuple(pltpu.VMEM((CHUNK, D), jnp.float32) for _ in range(NBUF)),
        pltpu.VMEM((SROWS, D), jnp.float32),
        pltpu.VMEM_SHARED((N, D), jnp.float32),
        tuple(pltpu.SemaphoreType.DMA for _ in range(NBUF)),
        tuple(pltpu.SemaphoreType.DMA for _ in range(NBUF)),
    ))
def _sc_agg(src_hbm, dst_hbm, m_hbm, agg_out,
            srcb, dstb, rowb, stage_v, agg_sh, gsem, ssem):
    c = lax.axis_index("c")
    s = lax.axis_index("s")
    ebase = c * EPC + s * EPS
    NB = NBUF

    _zero_fill(stage_v, SROWS)
    _zero_shared(stage_v, agg_sh, s)
    plsc.subcore_barrier()

    def _load_fire(j, b):
        base = pl.multiple_of(ebase + j * CHUNK, 8)
        pltpu.sync_copy(src_hbm.at[pl.ds(base, CHUNK)], srcb[b])
        pltpu.sync_copy(dst_hbm.at[pl.ds(base, CHUNK)], dstb[b])
        pltpu.async_copy(m_hbm.at[srcb[b]], rowb[b], gsem[b])

    for b in range(NB):
        _load_fire(b, b)

    def step(i, carry):
        for b in range(NB):
            pltpu.make_async_copy(m_hbm.at[srcb[b]], rowb[b], gsem[b]).wait()
            pltpu.async_copy(rowb[b], agg_sh.at[dstb[b]], ssem[b], add=True)
        for b in range(NB):
            pltpu.make_async_copy(rowb[b], agg_sh.at[dstb[b]], ssem[b]).wait()
            nj = i * NB + b + NB

            @pl.when(nj < NCHUNK)
            def _():
                _load_fire(nj, b)
        return carry

    lax.fori_loop(0, NCHUNK // NB, step, 0)
    for t in range(NCHUNK % NB):
        pltpu.make_async_copy(m_hbm.at[srcb[t]], rowb[t], gsem[t]).wait()
        pltpu.sync_copy(rowb[t], agg_sh.at[dstb[t]], add=True)
    plsc.subcore_barrier()
    _write_back(stage_v, agg_sh, agg_out, c, s)


@functools.partial(
    pl.kernel,
    out_type=jax.ShapeDtypeStruct((NC * N, D), jnp.float32),
    mesh=_MESH,
    scratch_types=(
        tuple(pltpu.VMEM((CHUNK,), jnp.int32) for _ in range(NBUF)),
        pltpu.VMEM((CHUNK, D), jnp.float32),
        pltpu.VMEM((SROWS, D), jnp.float32),
        pltpu.VMEM_SHARED((N, D), jnp.float32),
        tuple(pltpu.SemaphoreType.DMA for _ in range(NBUF)),
    ))
def _sc_deg(dst_hbm, deg_out, dstb, ones_v, stage_v, deg_sh, ssem):
    c = lax.axis_index("c")
    s = lax.axis_index("s")
    ebase = c * EPC + s * EPS
    NB = NBUF

    def of(i, carry):
        for g in range(D // 16):
            ones_v[i, pl.ds(g * 16, 16)] = jnp.ones((16,), jnp.float32)
        return carry
    lax.fori_loop(0, CHUNK, of, 0)

    _zero_fill(stage_v, SROWS)
    _zero_shared(stage_v, deg_sh, s)
    plsc.subcore_barrier()

    def _load_fire(j, b):
        base = pl.multiple_of(ebase + j * CHUNK, 8)
        pltpu.sync_copy(dst_hbm.at[pl.ds(base, CHUNK)], dstb[b])
        pltpu.async_copy(ones_v, deg_sh.at[dstb[b]], ssem[b], add=True)

    for b in range(NB):
        _load_fire(b, b)

    def step(i, carry):
        for b in range(NB):
            pltpu.make_async_copy(ones_v, deg_sh.at[dstb[b]], ssem[b]).wait()
            nj = i * NB + b + NB

            @pl.when(nj < NCHUNK)
            def _():
                _load_fire(nj, b)
        return carry

    lax.fori_loop(0, NCHUNK // NB, step, 0)
    for t in range(NCHUNK % NB):
        pltpu.make_async_copy(ones_v, deg_sh.at[dstb[t]], ssem[t]).wait()
    plsc.subcore_barrier()
    _write_back(stage_v, deg_sh, deg_out, c, s)


BR = 2000
GB = N // BR


def _mm_body(x_ref, w_ref, b_ref, m_ref):
    m_ref[...] = (jnp.dot(x_ref[...], w_ref[...],
                          preferred_element_type=jnp.float32) + b_ref[...])


def _tc_mm(x, W, b):
    return pl.pallas_call(
        _mm_body,
        grid=(GB,),
        in_specs=[pl.BlockSpec((BR, D), lambda i: (i, 0)),
                  pl.BlockSpec((D, D), lambda i: (0, 0)),
                  pl.BlockSpec((1, D), lambda i: (0, 0))],
        out_specs=pl.BlockSpec((BR, D), lambda i: (i, 0)),
        out_shape=jax.ShapeDtypeStruct((N, D), jnp.float32),
    )(x, W, b.reshape(1, D))


def _combine(h_ref, a_ref, g_ref):
    agg = a_ref[0] + a_ref[1]
    deg = jnp.maximum(g_ref[0, :, 0:1] + g_ref[1, :, 0:1], 1.0)
    return h_ref[...] + jnp.maximum(agg / deg, 0.0)


def _combine_mm_body(h_ref, a_ref, g_ref, w_ref, b_ref, hn_ref, m_ref):
    hn = _combine(h_ref, a_ref, g_ref)
    hn_ref[...] = hn
    m_ref[...] = (jnp.dot(hn, w_ref[...],
                          preferred_element_type=jnp.float32) + b_ref[...])


def _tc_combine_mm(h, agg2, deg2, W, b):
    return pl.pallas_call(
        _combine_mm_body,
        grid=(GB,),
        in_specs=[pl.BlockSpec((BR, D), lambda i: (i, 0)),
                  pl.BlockSpec((2, BR, D), lambda i: (0, i, 0)),
                  pl.BlockSpec((2, BR, D), lambda i: (0, i, 0)),
                  pl.BlockSpec((D, D), lambda i: (0, 0)),
                  pl.BlockSpec((1, D), lambda i: (0, 0))],
        out_specs=[pl.BlockSpec((BR, D), lambda i: (i, 0)),
                   pl.BlockSpec((BR, D), lambda i: (i, 0))],
        out_shape=[jax.ShapeDtypeStruct((N, D), jnp.float32),
                   jax.ShapeDtypeStruct((N, D), jnp.float32)],
    )(h, agg2, deg2, W, b.reshape(1, D))


def _combine_norm_body(h_ref, a_ref, g_ref, o_ref):
    hn = _combine(h_ref, a_ref, g_ref)
    nrm = jnp.sqrt(jnp.sum(hn * hn, axis=-1, keepdims=True))
    o_ref[...] = hn / jnp.maximum(nrm, 1e-12)


def _tc_combine_norm(h, agg2, deg2):
    return pl.pallas_call(
        _combine_norm_body,
        grid=(GB,),
        in_specs=[pl.BlockSpec((BR, D), lambda i: (i, 0)),
                  pl.BlockSpec((2, BR, D), lambda i: (0, i, 0)),
                  pl.BlockSpec((2, BR, D), lambda i: (0, i, 0))],
        out_specs=pl.BlockSpec((BR, D), lambda i: (i, 0)),
        out_shape=jax.ShapeDtypeStruct((N, D), jnp.float32),
    )(h, agg2, deg2)


def kernel(x, edge_index, W0, b0, W1, b1, W2, b2):
    src = edge_index[0]
    dst = edge_index[1]

    deg2 = _sc_deg(dst).reshape(2, N, D)
    m0 = _tc_mm(x, W0, b0)
    agg0 = _sc_agg(src, dst, m0).reshape(2, N, D)

    h1, m1 = _tc_combine_mm(x, agg0, deg2, W1, b1)
    agg1 = _sc_agg(src, dst, m1).reshape(2, N, D)

    h2, m2 = _tc_combine_mm(h1, agg1, deg2, W2, b2)
    agg2 = _sc_agg(src, dst, m2).reshape(2, N, D)

    return _tc_combine_norm(h2, agg2, deg2)
